# baseline (speedup 1.0000x reference)
import functools

import jax
import jax.numpy as jnp
from jax import lax
from jax.experimental import pallas as pl
from jax.experimental.pallas import tpu as pltpu
from jax.experimental.pallas import tpu_sc as plsc


def _build_lookup(BQ, H, V, D, C, NBUF):
    info = plsc.get_sparse_core_info()
    NC, NS = info.num_cores, info.num_subcores
    NW = NC * NS
    rows_w = BQ // NW
    b_per_w = rows_w * H
    rows_c = C // H
    n_chunks = b_per_w // C
    assert BQ % NW == 0 and C % H == 0 and b_per_w % C == 0
    assert n_chunks % NBUF == 0
    mesh = plsc.VectorSubcoreMesh(core_axis_name="c", subcore_axis_name="s")

    @functools.partial(
        pl.kernel,
        mesh=mesh,
        out_type=jax.ShapeDtypeStruct((BQ, H, D), jnp.float32),
        scratch_types=(
            [
                pltpu.VMEM((rows_w, H), jnp.int32),
                pltpu.VMEM((NBUF, H, D), jnp.float32),
            ]
            + [pltpu.SemaphoreType.DMA] * (2 * NBUF)
        ),
        compiler_params=pltpu.CompilerParams(use_tc_tiling_on_sc=False),
    )
    def lookup_kernel(table_hbm, idx_hbm, out_hbm, idx_v, rows_v, *sems):
        sg = sems[:NBUF]
        sw = sems[NBUF:]
        wid = lax.axis_index("s") * NC + lax.axis_index("c")
        row0 = wid * rows_w
        pltpu.sync_copy(idx_hbm.at[pl.ds(row0, rows_w)], idx_v)

        def start_gather(i, b):
            pltpu.async_copy(
                table_hbm.at[idx_v.at[i]],
                rows_v.at[b],
                sg[b],
            )

        def out_chunk(i):
            return out_hbm.at[row0 + i]

        for b in range(NBUF):
            start_gather(b, b)

        def outer(t, carry):
            g = t * NBUF
            for b in range(NBUF):
                i = g + b
                pltpu.make_async_copy(
                    out_chunk(0), rows_v.at[b], sg[b]
                ).wait()
                pltpu.async_copy(rows_v.at[b], out_chunk(i), sw[b])
                pltpu.make_async_copy(
                    rows_v.at[b], out_chunk(0), sw[b]
                ).wait()

                nxt = i + NBUF

                @pl.when(nxt < n_chunks)
                def _():
                    start_gather(nxt, b)

            return carry

        lax.fori_loop(0, n_chunks // NBUF, outer, 0)

    return lookup_kernel


@jax.jit
def kernel(token_idxs, table):
    BQ, H = token_idxs.shape
    V, D = table.shape
    return _build_lookup(BQ, H, V, D, 200, 8)(table, token_idxs)
